# baseline (speedup 1.0000x reference)
import functools

import jax
import jax.numpy as jnp
from jax import lax
from jax.experimental import pallas as pl
from jax.experimental.pallas import tpu as pltpu
from jax.experimental.pallas import tpu_sc as plsc

_NBUF = 2


def _make_gather(B: int, L: int, N: int, D: int):
    info = plsc.get_sparse_core_info()
    NC, NS = info.num_cores, info.num_subcores
    NW = NC * NS
    DH = D // 8
    BH = B // 128
    blocks = L * BH
    assert blocks % (NW * _NBUF) == 0
    bp = blocks // NW
    n_groups = bp // _NBUF

    mesh = plsc.VectorSubcoreMesh(core_axis_name="c", subcore_axis_name="s")

    @functools.partial(
        pl.kernel,
        mesh=mesh,
        out_type=jax.ShapeDtypeStruct((L, DH, BH, 8, 128), jnp.float32),
        scratch_types=[
            pltpu.VMEM((_NBUF, 128), jnp.int32),
            pltpu.VMEM((_NBUF, 128, D), jnp.float32),
            pltpu.VMEM((DH, 8, 128), jnp.float32),
            pltpu.VMEM((DH, 8, 128), jnp.float32),
            pltpu.SemaphoreType.DMA((_NBUF,)),
            pltpu.SemaphoreType.DMA((_NBUF,)),
            pltpu.SemaphoreType.DMA((_NBUF,)),
        ],
        compiler_params=pltpu.CompilerParams(
            use_tc_tiling_on_sc=False, needs_layout_passes=False
        ),
    )
    def gather_kernel(table_hbm, xt_hbm, out_hbm, idx_v, rows_v, tile_v0,
                      tile_v1, sem_i, sem_g, sem_s):
        tiles = [tile_v0, tile_v1]
        wid = lax.axis_index("s") * NC + lax.axis_index("c")
        base = wid * bp

        iota = lax.iota(jnp.int32, 16)
        dh_c = [(16 * k + iota) // 8 for k in range(D // 16)]
        dl_c = [(16 * k + iota) % 8 for k in range(D // 16)]

        def start_idx(i, b):
            blk = base + i
            l = blk // BH
            bh = blk % BH
            pltpu.async_copy(
                xt_hbm.at[l, pl.ds(bh * 128, 128)], idx_v.at[b], sem_i.at[b]
            )

        def start_gather(b):
            pltpu.async_copy(table_hbm.at[idx_v.at[b]], rows_v.at[b], sem_g.at[b])

        def wait_idx(b):
            pltpu.make_async_copy(
                xt_hbm.at[0, pl.ds(0, 128)], idx_v.at[b], sem_i.at[b]
            ).wait()

        def wait_gather(b):
            pltpu.make_async_copy(
                table_hbm.at[idx_v.at[b]], rows_v.at[b], sem_g.at[b]
            ).wait()

        def wait_store(b):
            pltpu.make_async_copy(
                tiles[b], out_hbm.at[0, :, 0], sem_s.at[b]
            ).wait()

        def transpose(b):
            for d in range(D):
                dv = jnp.full((16,), d, dtype=jnp.int32)
                for j in range(128 // 16):
                    v = plsc.load_gather(rows_v.at[b], [16 * j + iota, dv])
                    tiles[b][d // 8, d % 8, pl.ds(16 * j, 16)] = v

        def store(i, b):
            blk = base + i
            l = blk // BH
            bh = blk % BH
            pltpu.async_copy(tiles[b], out_hbm.at[l, :, bh], sem_s.at[b])

        for b in range(_NBUF):
            start_idx(b, b)
        for b in range(_NBUF):
            wait_idx(b)
            start_gather(b)

        def group(g, _):
            for b in range(_NBUF):
                i = g * _NBUF + b
                wait_gather(b)

                @pl.when(g < n_groups - 1)
                def _prefetch_idx():
                    start_idx(i + _NBUF, b)

                @pl.when(g > 0)
                def _drain_store():
                    wait_store(b)

                transpose(b)
                store(i, b)

                @pl.when(g < n_groups - 1)
                def _next_gather():
                    wait_idx(b)
                    start_gather(b)

            return 0

        lax.fori_loop(0, n_groups, group, 0)
        for b in range(_NBUF):
            wait_store(b)

    return gather_kernel


def kernel(x, table):
    B, L = x.shape
    N, D = table.shape
    xt = x.T
    out5 = _make_gather(B, L, N, D)(table, xt)
    perm = jnp.transpose(out5, (2, 4, 0, 1, 3))
    return perm.reshape(B, L, D)


# speedup vs baseline: 2.0027x; 2.0027x over previous
import functools

import jax
import jax.numpy as jnp
from jax import lax
from jax.experimental import pallas as pl
from jax.experimental.pallas import tpu as pltpu
from jax.experimental.pallas import tpu_sc as plsc

_NBUF = 2


def _make_gather(B: int, L: int, N: int, D: int):
    info = plsc.get_sparse_core_info()
    NC, NS = info.num_cores, info.num_subcores
    NW = NC * NS
    DH = D // 8
    BH = B // 128
    blocks = L * BH
    assert blocks % (NW * _NBUF) == 0
    bp = blocks // NW
    n_groups = bp // _NBUF

    mesh = plsc.VectorSubcoreMesh(core_axis_name="c", subcore_axis_name="s")

    @functools.partial(
        pl.kernel,
        mesh=mesh,
        out_type=jax.ShapeDtypeStruct((L, DH, BH, 8, 128), jnp.float32),
        scratch_types=[
            pltpu.VMEM((_NBUF, 128), jnp.int32),
            pltpu.VMEM((_NBUF, 128, D), jnp.float32),
            pltpu.VMEM((DH, 8, 128), jnp.float32),
            pltpu.VMEM((DH, 8, 128), jnp.float32),
            pltpu.SemaphoreType.DMA((_NBUF,)),
            pltpu.SemaphoreType.DMA((_NBUF,)),
            pltpu.SemaphoreType.DMA((_NBUF,)),
        ],
        compiler_params=pltpu.CompilerParams(
            use_tc_tiling_on_sc=False, needs_layout_passes=False
        ),
    )
    def gather_kernel(table_hbm, xt_hbm, out_hbm, idx_v, rows_v, tile_v0,
                      tile_v1, sem_i, sem_g, sem_s):
        tiles = [tile_v0, tile_v1]
        wid = lax.axis_index("s") * NC + lax.axis_index("c")
        base = wid * bp

        iota = lax.iota(jnp.int32, 16)
        dh_c = [(16 * k + iota) // 8 for k in range(D // 16)]
        dl_c = [(16 * k + iota) % 8 for k in range(D // 16)]

        def start_idx(i, b):
            blk = base + i
            l = blk // BH
            bh = blk % BH
            pltpu.async_copy(
                xt_hbm.at[l, pl.ds(bh * 128, 128)], idx_v.at[b], sem_i.at[b]
            )

        def start_gather(b):
            pltpu.async_copy(table_hbm.at[idx_v.at[b]], rows_v.at[b], sem_g.at[b])

        def wait_idx(b):
            pltpu.make_async_copy(
                xt_hbm.at[0, pl.ds(0, 128)], idx_v.at[b], sem_i.at[b]
            ).wait()

        def wait_gather(b):
            pltpu.make_async_copy(
                table_hbm.at[idx_v.at[b]], rows_v.at[b], sem_g.at[b]
            ).wait()

        def wait_store(b):
            pltpu.make_async_copy(
                tiles[b], out_hbm.at[0, :, 0], sem_s.at[b]
            ).wait()

        def transpose(b):
            @plsc.parallel_loop(0, D, unroll=8)
            def _col(d):
                dv = jnp.full((16,), d, dtype=jnp.int32)
                for j in range(128 // 16):
                    v = plsc.load_gather(rows_v.at[b], [16 * j + iota, dv])
                    tiles[b][d // 8, d % 8, pl.ds(16 * j, 16)] = v

        def store(i, b):
            blk = base + i
            l = blk // BH
            bh = blk % BH
            pltpu.async_copy(tiles[b], out_hbm.at[l, :, bh], sem_s.at[b])

        for b in range(_NBUF):
            start_idx(b, b)
        for b in range(_NBUF):
            wait_idx(b)
            start_gather(b)

        def group(g, _):
            for b in range(_NBUF):
                i = g * _NBUF + b
                wait_gather(b)

                @pl.when(g < n_groups - 1)
                def _prefetch_idx():
                    start_idx(i + _NBUF, b)

                @pl.when(g > 0)
                def _drain_store():
                    wait_store(b)

                transpose(b)
                store(i, b)

                @pl.when(g < n_groups - 1)
                def _next_gather():
                    wait_idx(b)
                    start_gather(b)

            return 0

        lax.fori_loop(0, n_groups, group, 0)
        for b in range(_NBUF):
            wait_store(b)

    return gather_kernel


def kernel(x, table):
    B, L = x.shape
    N, D = table.shape
    xt = x.T
    out5 = _make_gather(B, L, N, D)(table, xt)
    perm = jnp.transpose(out5, (2, 4, 0, 1, 3))
    return perm.reshape(B, L, D)


# speedup vs baseline: 5.6019x; 2.7971x over previous
import functools

import jax
import jax.numpy as jnp
from jax import lax
from jax.experimental import pallas as pl
from jax.experimental.pallas import tpu as pltpu
from jax.experimental.pallas import tpu_sc as plsc

_NBUF = 2


def _make_gather(B: int, L: int, N: int, D: int):
    info = plsc.get_sparse_core_info()
    NC, NS = info.num_cores, info.num_subcores
    NW = NC * NS
    DH = D // 8
    BH = B // 128
    blocks = L * BH
    assert blocks % (NW * _NBUF) == 0
    bp = blocks // NW
    n_groups = bp // _NBUF

    mesh = plsc.VectorSubcoreMesh(core_axis_name="c", subcore_axis_name="s")

    @functools.partial(
        pl.kernel,
        mesh=mesh,
        out_type=jax.ShapeDtypeStruct((L, DH, BH, 8, 128), jnp.float32),
        scratch_types=[
            pltpu.VMEM((_NBUF, 128), jnp.int32),
            pltpu.VMEM((_NBUF, 128, D), jnp.float32),
            pltpu.VMEM((DH, 8, 129), jnp.float32),
            pltpu.VMEM((DH, 8, 129), jnp.float32),
            pltpu.SemaphoreType.DMA((_NBUF,)),
            pltpu.SemaphoreType.DMA((_NBUF,)),
            pltpu.SemaphoreType.DMA((_NBUF,)),
        ],
        compiler_params=pltpu.CompilerParams(
            use_tc_tiling_on_sc=False, needs_layout_passes=False
        ),
    )
    def gather_kernel(table_hbm, xt_hbm, out_hbm, idx_v, rows_v, tile_v0,
                      tile_v1, sem_i, sem_g, sem_s):
        tiles = [tile_v0, tile_v1]
        wid = lax.axis_index("s") * NC + lax.axis_index("c")
        base = wid * bp

        iota = lax.iota(jnp.int32, 16)
        dh_c = [(16 * k + iota) // 8 for k in range(D // 16)]
        dl_c = [(16 * k + iota) % 8 for k in range(D // 16)]

        def start_idx(i, b):
            blk = base + i
            l = blk // BH
            bh = blk % BH
            pltpu.async_copy(
                xt_hbm.at[l, pl.ds(bh * 128, 128)], idx_v.at[b], sem_i.at[b]
            )

        def start_gather(b):
            pltpu.async_copy(table_hbm.at[idx_v.at[b]], rows_v.at[b], sem_g.at[b])

        def wait_idx(b):
            pltpu.make_async_copy(
                xt_hbm.at[0, pl.ds(0, 128)], idx_v.at[b], sem_i.at[b]
            ).wait()

        def wait_gather(b):
            pltpu.make_async_copy(
                table_hbm.at[idx_v.at[b]], rows_v.at[b], sem_g.at[b]
            ).wait()

        def wait_store(b):
            pltpu.make_async_copy(
                tiles[b].at[:, :, pl.ds(0, 128)], out_hbm.at[0, :, 0], sem_s.at[b]
            ).wait()

        def transpose(b):
            @plsc.parallel_loop(0, 128, unroll=8)
            def _row(r):
                rv = jnp.full((16,), r, dtype=jnp.int32)
                for k in range(D // 16):
                    v = rows_v[b, r, pl.ds(16 * k, 16)]
                    plsc.store_scatter(tiles[b], [dh_c[k], dl_c[k], rv], v)

        def store(i, b):
            blk = base + i
            l = blk // BH
            bh = blk % BH
            pltpu.async_copy(
                tiles[b].at[:, :, pl.ds(0, 128)], out_hbm.at[l, :, bh], sem_s.at[b]
            )

        for b in range(_NBUF):
            start_idx(b, b)
        for b in range(_NBUF):
            wait_idx(b)
            start_gather(b)

        def group(g, _):
            for b in range(_NBUF):
                i = g * _NBUF + b
                wait_gather(b)

                @pl.when(g < n_groups - 1)
                def _prefetch_idx():
                    start_idx(i + _NBUF, b)

                @pl.when(g > 0)
                def _drain_store():
                    wait_store(b)

                transpose(b)
                store(i, b)

                @pl.when(g < n_groups - 1)
                def _next_gather():
                    wait_idx(b)
                    start_gather(b)

            return 0

        lax.fori_loop(0, n_groups, group, 0)
        for b in range(_NBUF):
            wait_store(b)

    return gather_kernel


def kernel(x, table):
    B, L = x.shape
    N, D = table.shape
    xt = x.T
    out5 = _make_gather(B, L, N, D)(table, xt)
    perm = jnp.transpose(out5, (2, 4, 0, 1, 3))
    return perm.reshape(B, L, D)


# speedup vs baseline: 6.5071x; 1.1616x over previous
import functools

import jax
import jax.numpy as jnp
from jax import lax
from jax.experimental import pallas as pl
from jax.experimental.pallas import tpu as pltpu
from jax.experimental.pallas import tpu_sc as plsc

_NBUF = 4


def _make_gather(B: int, L: int, N: int, D: int):
    info = plsc.get_sparse_core_info()
    NC, NS = info.num_cores, info.num_subcores
    NW = NC * NS
    DH = D // 8
    BH = B // 128
    blocks = L * BH
    assert blocks % (NW * _NBUF) == 0
    bp = blocks // NW
    n_groups = bp // _NBUF

    mesh = plsc.VectorSubcoreMesh(core_axis_name="c", subcore_axis_name="s")

    @functools.partial(
        pl.kernel,
        mesh=mesh,
        out_type=jax.ShapeDtypeStruct((L, DH, BH, 8, 128), jnp.float32),
        scratch_types=[
            pltpu.VMEM((_NBUF, 128), jnp.int32),
            pltpu.VMEM((_NBUF, 128, D), jnp.float32),
            pltpu.VMEM((DH, 8, 129), jnp.float32),
            pltpu.VMEM((DH, 8, 129), jnp.float32),
            pltpu.VMEM((DH, 8, 129), jnp.float32),
            pltpu.VMEM((DH, 8, 129), jnp.float32),
            pltpu.SemaphoreType.DMA((_NBUF,)),
            pltpu.SemaphoreType.DMA((_NBUF,)),
            pltpu.SemaphoreType.DMA((_NBUF,)),
        ],
        compiler_params=pltpu.CompilerParams(
            use_tc_tiling_on_sc=False, needs_layout_passes=False
        ),
    )
    def gather_kernel(table_hbm, xt_hbm, out_hbm, idx_v, rows_v, tile_v0,
                      tile_v1, tile_v2, tile_v3, sem_i, sem_g, sem_s):
        tiles = [tile_v0, tile_v1, tile_v2, tile_v3]
        wid = lax.axis_index("s") * NC + lax.axis_index("c")
        base = wid * bp

        iota = lax.iota(jnp.int32, 16)
        dh_c = [(16 * k + iota) // 8 for k in range(D // 16)]
        dl_c = [(16 * k + iota) % 8 for k in range(D // 16)]

        def start_idx(i, b):
            blk = base + i
            l = blk // BH
            bh = blk % BH
            pltpu.async_copy(
                xt_hbm.at[l, pl.ds(bh * 128, 128)], idx_v.at[b], sem_i.at[b]
            )

        def start_gather(b):
            pltpu.async_copy(table_hbm.at[idx_v.at[b]], rows_v.at[b], sem_g.at[b])

        def wait_idx(b):
            pltpu.make_async_copy(
                xt_hbm.at[0, pl.ds(0, 128)], idx_v.at[b], sem_i.at[b]
            ).wait()

        def wait_gather(b):
            pltpu.make_async_copy(
                table_hbm.at[idx_v.at[b]], rows_v.at[b], sem_g.at[b]
            ).wait()

        def wait_store(b):
            pltpu.make_async_copy(
                tiles[b].at[:, :, pl.ds(0, 128)], out_hbm.at[0, :, 0], sem_s.at[b]
            ).wait()

        def transpose(b):
            @plsc.parallel_loop(0, 128, unroll=8)
            def _row(r):
                rv = jnp.full((16,), r, dtype=jnp.int32)
                for k in range(D // 16):
                    v = rows_v[b, r, pl.ds(16 * k, 16)]
                    plsc.store_scatter(tiles[b], [dh_c[k], dl_c[k], rv], v)

        def store(i, b):
            blk = base + i
            l = blk // BH
            bh = blk % BH
            pltpu.async_copy(
                tiles[b].at[:, :, pl.ds(0, 128)], out_hbm.at[l, :, bh], sem_s.at[b]
            )

        for b in range(_NBUF):
            start_idx(b, b)
        for b in range(_NBUF):
            wait_idx(b)
            start_gather(b)

        def group(g, _):
            for b in range(_NBUF):
                i = g * _NBUF + b
                wait_gather(b)

                @pl.when(g < n_groups - 1)
                def _prefetch_idx():
                    start_idx(i + _NBUF, b)

                @pl.when(g > 0)
                def _drain_store():
                    wait_store(b)

                transpose(b)
                store(i, b)

                @pl.when(g < n_groups - 1)
                def _next_gather():
                    wait_idx(b)
                    start_gather(b)

            return 0

        lax.fori_loop(0, n_groups, group, 0)
        for b in range(_NBUF):
            wait_store(b)

    return gather_kernel


def kernel(x, table):
    B, L = x.shape
    N, D = table.shape
    xt = x.T
    out5 = _make_gather(B, L, N, D)(table, xt)
    perm = jnp.transpose(out5, (2, 4, 0, 1, 3))
    return perm.reshape(B, L, D)
